# baseline (speedup 1.0000x reference)
import functools

import jax
import jax.numpy as jnp
from jax.experimental import pallas as pl
from jax.experimental.pallas import tpu as pltpu

BN_EPS = 1e-5
NUM_CLASSES = 13
SA_CFG = [
    (4096, (0.1, 0.2), (32, 64)),
    (2048, (0.2, 0.4), (32, 64)),
    (512, (0.4, 0.8), (32, 64)),
    (128, (0.8, 1.6), (32, 64)),
]


def _sqdist(src, dst):
    return (jnp.sum(src ** 2, -1)[:, :, None] + jnp.sum(dst ** 2, -1)[:, None, :]
            - 2.0 * jnp.einsum('bnc,bmc->bnm', src, dst))


def _index_points(points, idx):
    b = points.shape[0]
    bidx = jnp.arange(b).reshape((b,) + (1,) * (idx.ndim - 1))
    return points[bidx, idx]


_FPS_STAGES = (4096, 2048, 512, 128)


def _fps_stage_body(xyz_ref, o_ref, *, S):
    R = xyz_ref.shape[1]
    n = R * 128
    xr = xyz_ref[0]
    yr = xyz_ref[1]
    zr = xyz_ref[2]
    gi = (jax.lax.broadcasted_iota(jnp.int32, (R, 128), 0) * 128
          + jax.lax.broadcasted_iota(jnp.int32, (R, 128), 1))
    RS = S // 128
    gs = (jax.lax.broadcasted_iota(jnp.int32, (RS, 128), 0) * 128
          + jax.lax.broadcasted_iota(jnp.int32, (RS, 128), 1))

    def body(i, state):
        dist, far, idx_out = state
        sel = gi == far
        cx = jnp.sum(jnp.where(sel, xr, 0.0))
        cy = jnp.sum(jnp.where(sel, yr, 0.0))
        cz = jnp.sum(jnp.where(sel, zr, 0.0))
        dx = xr - cx
        dy = yr - cy
        dz = zr - cz
        d = dx * dx + dy * dy + dz * dz
        dist = jnp.minimum(dist, d)
        m = jnp.max(dist)
        far_new = jnp.min(jnp.where(dist == m, gi, n))
        idx_out = jnp.where(gs == i, far, idx_out)
        return dist, far_new, idx_out

    init = (jnp.full((R, 128), 1e10, jnp.float32), jnp.int32(0),
            jnp.zeros((RS, 128), jnp.int32))
    _, _, idx_out = jax.lax.fori_loop(0, S, body, init)
    o_ref[...] = idx_out


def _fps_chain(xyz_norm_t):
    cur = xyz_norm_t
    idx = None
    out = []
    for S in _FPS_STAGES:
        n = cur.shape[0]
        x3 = cur.T.reshape(3, n // 128, 128)
        loc = pl.pallas_call(
            functools.partial(_fps_stage_body, S=S),
            out_shape=jax.ShapeDtypeStruct((S // 128, 128), jnp.int32),
        )(x3).reshape(-1)
        gidx = loc if idx is None else idx[loc]
        out.append(gidx)
        cur = cur[loc]
        idx = gidx
    return tuple(out)


def _ball_query_body(q_ref, x_ref, o1_ref, o2_ref, c_scr, rks):
    x3 = x_ref[...]
    q = q_ref[...]
    n = x3.shape[1]
    qb = q.shape[0]
    qn = jnp.sum(q * q, axis=1, keepdims=True)
    xn = jnp.sum(x3 * x3, axis=0, keepdims=True)
    dist = qn + xn - 2.0 * jax.lax.dot(q, x3, preferred_element_type=jnp.float32)
    tri = (jax.lax.broadcasted_iota(jnp.int32, (128, 128), 0)
           <= jax.lax.broadcasted_iota(jnp.int32, (128, 128), 1)).astype(jnp.float32)
    for (radius, k, oref) in ((rks[0][0], rks[0][1], o1_ref), (rks[1][0], rks[1][1], o2_ref)):
        maskf = jnp.where(dist <= radius * radius, 1.0, 0.0)
        off = jnp.zeros((qb, 1), jnp.float32)
        for m in range(n // 128):
            cm = jax.lax.dot(maskf[:, m * 128:(m + 1) * 128], tri,
                             preferred_element_type=jnp.float32)
            c_scr[:, m * 128:(m + 1) * 128] = cm + off
            off = off + cm[:, 127:128]
        c = c_scr[...]
        cols = []
        for r in range(k):
            cols.append(jnp.sum(jnp.where(c < float(r + 1), 1.0, 0.0), axis=1,
                                keepdims=True))
        cnt = jnp.concatenate(cols, axis=1)
        first = cols[0]
        oref[...] = jnp.where(cnt >= float(n), first, cnt).astype(jnp.int32)


def _ball_query2(radii, nsamples, xyz_t, new_xyz_t):
    n = xyz_t.shape[0]
    s = new_xyz_t.shape[0]
    qb = min(256, s)
    k1, k2 = nsamples
    body = functools.partial(_ball_query_body, rks=((radii[0], k1), (radii[1], k2)))
    o1, o2 = pl.pallas_call(
        body,
        grid=(s // qb,),
        in_specs=[
            pl.BlockSpec((qb, 3), lambda i: (i, 0)),
            pl.BlockSpec((3, n), lambda i: (0, 0)),
        ],
        out_specs=[
            pl.BlockSpec((qb, k1), lambda i: (i, 0)),
            pl.BlockSpec((qb, k2), lambda i: (i, 0)),
        ],
        out_shape=[
            jax.ShapeDtypeStruct((s, k1), jnp.int32),
            jax.ShapeDtypeStruct((s, k2), jnp.int32),
        ],
        scratch_shapes=[pltpu.VMEM((qb, n), jnp.float32)],
    )(new_xyz_t, xyz_t.T)
    return o1, o2


def _fold_bn(p):
    W, b, gamma, beta = p
    s = gamma / jnp.sqrt(1.0 + BN_EPS)
    return W * s[:, None], b * s + beta


def _sa_conv_body(g_ref, q_ref, *refs, k, cp):
    wb_refs, o_ref = refs[:-1], refs[-1]
    g = g_ref[...]
    sb = q_ref.shape[0]
    w1t = wb_refs[0][...]
    h = jax.lax.dot(g, w1t, preferred_element_type=jnp.float32) + wb_refs[1][...]
    off = jax.lax.dot(q_ref[...], w1t[cp:cp + 3, :],
                      preferred_element_type=jnp.float32)
    h = h.reshape(sb, k, -1) - off[:, None, :]
    h = jnp.maximum(h, 0.0).reshape(sb * k, -1)
    h = jax.lax.dot(h, wb_refs[2][...], preferred_element_type=jnp.float32) + wb_refs[3][...]
    h = jnp.maximum(h, 0.0)
    h = jax.lax.dot(h, wb_refs[4][...], preferred_element_type=jnp.float32) + wb_refs[5][...]
    h = jnp.maximum(h, 0.0)
    o_ref[...] = jnp.max(h.reshape(sb, k, -1), axis=1)


def _sa_branch_pallas(gathered, new_xyz_t, convs, k):
    sk, c = gathered.shape
    s = sk // k
    cp = c - 3
    (w1, b1), (w2, b2), (w3, b3) = (_fold_bn(p) for p in convs)
    o1, o2, o3 = w1.shape[0], w2.shape[0], w3.shape[0]
    rows_target = 2048 if sk >= 2048 else sk
    sb = max(1, rows_target // k)
    while s % sb:
        sb //= 2
    body = functools.partial(_sa_conv_body, k=k, cp=cp)
    wspecs = [
        pl.BlockSpec((c, o1), lambda i: (0, 0)),
        pl.BlockSpec((1, o1), lambda i: (0, 0)),
        pl.BlockSpec((o1, o2), lambda i: (0, 0)),
        pl.BlockSpec((1, o2), lambda i: (0, 0)),
        pl.BlockSpec((o2, o3), lambda i: (0, 0)),
        pl.BlockSpec((1, o3), lambda i: (0, 0)),
    ]
    out = pl.pallas_call(
        body,
        grid=(s // sb,),
        in_specs=[
            pl.BlockSpec((sb * k, c), lambda i: (i, 0)),
            pl.BlockSpec((sb, 3), lambda i: (i, 0)),
        ] + wspecs,
        out_specs=pl.BlockSpec((sb, o3), lambda i: (i, 0)),
        out_shape=jax.ShapeDtypeStruct((s, o3), jnp.float32),
    )(gathered, new_xyz_t, w1.T, b1[None], w2.T, b2[None], w3.T, b3[None])
    return out


def _conv_bn_relu(x, W, bias, gamma, beta):
    shp = (1, -1) + (1,) * (x.ndim - 2)
    y = jnp.einsum('oc,bc...->bo...', W, x) + bias.reshape(shp)
    y = y / jnp.sqrt(1.0 + BN_EPS)
    y = y * gamma.reshape(shp) + beta.reshape(shp)
    return jax.nn.relu(y)


def _sa_msg(xyz, points, new_xyz, radii, nsamples, branch_params):
    xyz_t = jnp.transpose(xyz, (0, 2, 1))
    points_t = None if points is None else jnp.transpose(points, (0, 2, 1))
    outs = []
    gidx_both = _ball_query2(radii, nsamples, xyz_t[0], new_xyz[0])
    table = jnp.concatenate([points_t[0], xyz_t[0]], axis=1)
    for k, convs, gidx in zip(nsamples, branch_params, gidx_both):
        gathered = table[gidx.reshape(-1)]
        out = _sa_branch_pallas(gathered, new_xyz[0], convs, k)
        outs.append(out.T[None])
    return jnp.transpose(new_xyz, (0, 2, 1)), jnp.concatenate(outs, axis=1)


def _fp_body(*refs, n_convs, has_p1):
    if has_p1:
        (x1_ref, x2t_ref, p2_ref, p1_ref), rest = refs[:4], refs[4:]
    else:
        (x1_ref, x2t_ref, p2_ref), rest = refs[:3], refs[3:]
        p1_ref = None
    wb_refs, o_ref = rest[:-1], rest[-1]
    x1 = x1_ref[...]
    x2t = x2t_ref[...]
    qb = x1.shape[0]
    s2 = x2t.shape[1]
    qn = jnp.sum(x1 * x1, axis=1, keepdims=True)
    xn = jnp.sum(x2t * x2t, axis=0, keepdims=True)
    d = qn + xn - 2.0 * jax.lax.dot(x1, x2t, preferred_element_type=jnp.float32)
    iota = jax.lax.broadcasted_iota(jnp.int32, (qb, s2), 1)
    acc = jnp.zeros((qb, s2), jnp.float32)
    recips = []
    onehots = []
    for _ in range(3):
        m = jnp.min(d, axis=1, keepdims=True)
        jmin = jnp.min(jnp.where(d == m, iota, s2), axis=1, keepdims=True)
        oh = (iota == jmin)
        onehots.append(oh)
        recips.append(1.0 / (m + 1e-8))
        d = jnp.where(oh, 1e30, d)
    norm = recips[0] + recips[1] + recips[2]
    for oh, rc in zip(onehots, recips):
        acc = acc + jnp.where(oh, rc / norm, 0.0)
    h = jax.lax.dot(acc, p2_ref[...], preferred_element_type=jnp.float32)
    if p1_ref is not None:
        h = jnp.concatenate([p1_ref[...], h], axis=1)
    for i in range(n_convs):
        h = jax.lax.dot(h, wb_refs[2 * i][...], preferred_element_type=jnp.float32)
        h = jnp.maximum(h + wb_refs[2 * i + 1][...], 0.0)
    o_ref[...] = h


def _fp_pallas(x1t, x2t, p1, p2, convs):
    s1 = x1t.shape[0]
    s2 = x2t.shape[0]
    c2 = p2.shape[1]
    qb = min(256, s1)
    wbs = [_fold_bn(p) for p in convs]
    n_convs = len(wbs)
    cout = wbs[-1][0].shape[0]
    in_specs = [
        pl.BlockSpec((qb, 3), lambda i: (i, 0)),
        pl.BlockSpec((3, s2), lambda i: (0, 0)),
        pl.BlockSpec((s2, c2), lambda i: (0, 0)),
    ]
    args = [x1t, x2t.T, p2]
    if p1 is not None:
        in_specs.append(pl.BlockSpec((qb, p1.shape[1]), lambda i: (i, 0)))
        args.append(p1)
    for (w, b) in wbs:
        in_specs.append(pl.BlockSpec(w.T.shape, lambda i: (0, 0)))
        in_specs.append(pl.BlockSpec((1, w.shape[0]), lambda i: (0, 0)))
        args.append(w.T)
        args.append(b[None])
    body = functools.partial(_fp_body, n_convs=n_convs, has_p1=p1 is not None)
    return pl.pallas_call(
        body,
        grid=(s1 // qb,),
        in_specs=in_specs,
        out_specs=pl.BlockSpec((qb, cout), lambda i: (i, 0)),
        out_shape=jax.ShapeDtypeStruct((s1, cout), jnp.float32),
    )(*args)


def _fp_module(xyz1, xyz2, points1, points2, convs):
    x1 = jnp.transpose(xyz1, (0, 2, 1))
    x2 = jnp.transpose(xyz2, (0, 2, 1))
    p2 = jnp.transpose(points2, (0, 2, 1))
    dists = _sqdist(x1, x2)
    idx = jnp.argsort(dists, axis=-1)[:, :, :3]
    d3 = jnp.take_along_axis(dists, idx, axis=-1)
    dist_recip = 1.0 / (d3 + 1e-8)
    weight = dist_recip / jnp.sum(dist_recip, axis=2, keepdims=True)
    interp = jnp.sum(_index_points(p2, idx) * weight[..., None], axis=2)
    if points1 is not None:
        newp = jnp.concatenate([jnp.transpose(points1, (0, 2, 1)), interp], axis=-1)
    else:
        newp = interp
    g = jnp.transpose(newp, (0, 2, 1))
    for p in convs:
        g = _conv_bn_relu(g, *p)
    return g


def _head_pallas(h, W2, b2):
    b, c, n = h.shape

    def kern(h_ref, w_ref, b_ref, o_ref):
        hh = h_ref[0]
        logits = jnp.dot(w_ref[...], hh, preferred_element_type=jnp.float32)
        logits = logits + b_ref[...][:, :1]
        m = jnp.max(logits, axis=0, keepdims=True)
        z = logits - m
        lse = jnp.log(jnp.sum(jnp.exp(z), axis=0, keepdims=True))
        o_ref[0] = z - lse

    blk = 2048
    out = pl.pallas_call(
        kern,
        grid=(n // blk,),
        in_specs=[
            pl.BlockSpec((1, c, blk), lambda i: (0, 0, i)),
            pl.BlockSpec((NUM_CLASSES, c), lambda i: (0, 0)),
            pl.BlockSpec((NUM_CLASSES, 1), lambda i: (0, 0)),
        ],
        out_specs=pl.BlockSpec((1, NUM_CLASSES, blk), lambda i: (0, 0, i)),
        out_shape=jax.ShapeDtypeStruct((b, NUM_CLASSES, n), jnp.float32),
    )(h, W2, b2.reshape(NUM_CLASSES, 1))
    return out


@jax.jit
def kernel(xyz, colors, sa_params, fp_params, head1, head2):
    del colors
    mean = jnp.mean(xyz, axis=2, keepdims=True)
    std = jnp.std(xyz, axis=2, keepdims=True, ddof=1)
    std = jnp.where(std == 0, 1e-8, std)
    x = (xyz - mean) / std
    l0_xyz, l0_points = x, x
    x_norm_t = x[0].T
    fps_idx = _fps_chain(x_norm_t)
    layers = []
    cur_xyz, cur_pts = l0_xyz, l0_points
    for cfg, prm, fidx in zip(SA_CFG, sa_params, fps_idx):
        new_xyz = x_norm_t[fidx][None]
        cur_xyz, cur_pts = _sa_msg(cur_xyz, cur_pts, new_xyz, cfg[1], cfg[2], prm)
        layers.append((cur_xyz, cur_pts))
    (l1_xyz, l1_points), (l2_xyz, l2_points), (l3_xyz, l3_points), (l4_xyz, l4_points) = layers

    def fp(xyz1, xyz2, points1, points2, convs):
        p1 = None if points1 is None else points1[0].T
        out = _fp_pallas(xyz1[0].T, xyz2[0].T, p1, points2[0].T, convs)
        return out.T[None]

    l3_points = fp(l3_xyz, l4_xyz, l3_points, l4_points, fp_params[0])
    l2_points = fp(l2_xyz, l3_xyz, l2_points, l3_points, fp_params[1])
    l1_points = fp(l1_xyz, l2_xyz, l1_points, l2_points, fp_params[2])
    l0_points = fp(l0_xyz, l1_xyz, None, l1_points, fp_params[3])
    h = _conv_bn_relu(l0_points, *head1)
    W2, b2 = head2
    logits = _head_pallas(h, W2, b2)
    return jnp.transpose(logits, (0, 2, 1))


# speedup vs baseline: 1.0254x; 1.0254x over previous
import functools

import jax
import jax.numpy as jnp
from jax.experimental import pallas as pl
from jax.experimental.pallas import tpu as pltpu

BN_EPS = 1e-5
NUM_CLASSES = 13
SA_CFG = [
    (4096, (0.1, 0.2), (32, 64)),
    (2048, (0.2, 0.4), (32, 64)),
    (512, (0.4, 0.8), (32, 64)),
    (128, (0.8, 1.6), (32, 64)),
]


def _sqdist(src, dst):
    return (jnp.sum(src ** 2, -1)[:, :, None] + jnp.sum(dst ** 2, -1)[:, None, :]
            - 2.0 * jnp.einsum('bnc,bmc->bnm', src, dst))


def _index_points(points, idx):
    b = points.shape[0]
    bidx = jnp.arange(b).reshape((b,) + (1,) * (idx.ndim - 1))
    return points[bidx, idx]


_FPS_STAGES = (4096, 2048, 512, 128)


def _fps_stage_body(xyz_ref, o_ref, *, S):
    R = xyz_ref.shape[1]
    n = R * 128
    xr = xyz_ref[0]
    yr = xyz_ref[1]
    zr = xyz_ref[2]
    gi = (jax.lax.broadcasted_iota(jnp.int32, (R, 128), 0) * 128
          + jax.lax.broadcasted_iota(jnp.int32, (R, 128), 1))
    RS = S // 128
    gs = (jax.lax.broadcasted_iota(jnp.int32, (RS, 128), 0) * 128
          + jax.lax.broadcasted_iota(jnp.int32, (RS, 128), 1))

    def body(i, state):
        dist, far, idx_out = state
        sel = gi == far
        cx = jnp.sum(jnp.where(sel, xr, 0.0))
        cy = jnp.sum(jnp.where(sel, yr, 0.0))
        cz = jnp.sum(jnp.where(sel, zr, 0.0))
        dx = xr - cx
        dy = yr - cy
        dz = zr - cz
        d = dx * dx + dy * dy + dz * dz
        dist = jnp.minimum(dist, d)
        m = jnp.max(dist)
        far_new = jnp.min(jnp.where(dist == m, gi, n))
        idx_out = jnp.where(gs == i, far, idx_out)
        return dist, far_new, idx_out

    init = (jnp.full((R, 128), 1e10, jnp.float32), jnp.int32(0),
            jnp.zeros((RS, 128), jnp.int32))
    _, _, idx_out = jax.lax.fori_loop(0, S, body, init)
    o_ref[...] = idx_out


def _fps_chain(xyz_norm_t):
    cur = xyz_norm_t
    idx = None
    out = []
    for S in _FPS_STAGES:
        n = cur.shape[0]
        x3 = cur.T.reshape(3, n // 128, 128)
        loc = pl.pallas_call(
            functools.partial(_fps_stage_body, S=S),
            out_shape=jax.ShapeDtypeStruct((S // 128, 128), jnp.int32),
        )(x3).reshape(-1)
        gidx = loc if idx is None else idx[loc]
        out.append(gidx)
        cur = cur[loc]
        idx = gidx
    return tuple(out)


def _ball_query_body(q_ref, x_ref, o1_ref, o2_ref, c_scr, rks):
    x3 = x_ref[...]
    q = q_ref[...]
    n = x3.shape[1]
    qb = q.shape[0]
    qn = jnp.sum(q * q, axis=1, keepdims=True)
    xn = jnp.sum(x3 * x3, axis=0, keepdims=True)
    dist = qn + xn - 2.0 * jax.lax.dot(q, x3, preferred_element_type=jnp.float32)
    tri = (jax.lax.broadcasted_iota(jnp.int32, (128, 128), 0)
           <= jax.lax.broadcasted_iota(jnp.int32, (128, 128), 1)).astype(jnp.float32)
    for (radius, k, oref) in ((rks[0][0], rks[0][1], o1_ref), (rks[1][0], rks[1][1], o2_ref)):
        maskf = jnp.where(dist <= radius * radius, 1.0, 0.0)
        off = jnp.zeros((qb, 1), jnp.float32)
        for m in range(n // 128):
            cm = jax.lax.dot(maskf[:, m * 128:(m + 1) * 128], tri,
                             preferred_element_type=jnp.float32)
            c_scr[:, m * 128:(m + 1) * 128] = cm + off
            off = off + cm[:, 127:128]
        c = c_scr[...]
        cols = []
        for r in range(k):
            cols.append(jnp.sum(jnp.where(c < float(r + 1), 1.0, 0.0), axis=1,
                                keepdims=True))
        cnt = jnp.concatenate(cols, axis=1)
        first = cols[0]
        oref[...] = jnp.where(cnt >= float(n), first, cnt).astype(jnp.int32)


def _ball_query2(radii, nsamples, xyz_t, new_xyz_t):
    n = xyz_t.shape[0]
    s = new_xyz_t.shape[0]
    qb = min(256, s)
    k1, k2 = nsamples
    body = functools.partial(_ball_query_body, rks=((radii[0], k1), (radii[1], k2)))
    o1, o2 = pl.pallas_call(
        body,
        grid=(s // qb,),
        in_specs=[
            pl.BlockSpec((qb, 3), lambda i: (i, 0)),
            pl.BlockSpec((3, n), lambda i: (0, 0)),
        ],
        out_specs=[
            pl.BlockSpec((qb, k1), lambda i: (i, 0)),
            pl.BlockSpec((qb, k2), lambda i: (i, 0)),
        ],
        out_shape=[
            jax.ShapeDtypeStruct((s, k1), jnp.int32),
            jax.ShapeDtypeStruct((s, k2), jnp.int32),
        ],
        scratch_shapes=[pltpu.VMEM((qb, n), jnp.float32)],
    )(new_xyz_t, xyz_t.T)
    return o1, o2


def _fold_bn(p):
    W, b, gamma, beta = p
    s = gamma / jnp.sqrt(1.0 + BN_EPS)
    return W * s[:, None], b * s + beta


def _pre_conv_body(t_ref, w_ref, b_ref, o_ref):
    o_ref[...] = (jax.lax.dot(t_ref[...], w_ref[...],
                              preferred_element_type=jnp.float32) + b_ref[...])


def _pre_conv_pallas(table, w1t, b1):
    n, c = table.shape
    o1 = w1t.shape[1]
    rb = min(1024, n)
    return pl.pallas_call(
        _pre_conv_body,
        grid=(n // rb,),
        in_specs=[
            pl.BlockSpec((rb, c), lambda i: (i, 0)),
            pl.BlockSpec((c, o1), lambda i: (0, 0)),
            pl.BlockSpec((1, o1), lambda i: (0, 0)),
        ],
        out_specs=pl.BlockSpec((rb, o1), lambda i: (i, 0)),
        out_shape=jax.ShapeDtypeStruct((n, o1), jnp.float32),
    )(table, w1t, b1[None])


def _sa_conv_body(g_ref, q_ref, wx_ref, w2_ref, b2_ref, w3_ref, b3_ref, o_ref,
                  *, k):
    sb = q_ref.shape[0]
    off = jax.lax.dot(q_ref[...], wx_ref[...],
                      preferred_element_type=jnp.float32)
    h = g_ref[...].reshape(sb, k, -1) - off[:, None, :]
    h = jnp.maximum(h, 0.0).reshape(sb * k, -1)
    h = jax.lax.dot(h, w2_ref[...], preferred_element_type=jnp.float32) + b2_ref[...]
    h = jnp.maximum(h, 0.0)
    h = jax.lax.dot(h, w3_ref[...], preferred_element_type=jnp.float32) + b3_ref[...]
    h = jnp.maximum(h, 0.0)
    o_ref[...] = jnp.max(h.reshape(sb, k, -1), axis=1)


def _sa_branch_pallas(table, gidx, new_xyz_t, convs, k):
    s = gidx.shape[0]
    cp = table.shape[1] - 3
    (w1, b1), (w2, b2), (w3, b3) = (_fold_bn(p) for p in convs)
    o1, o2, o3 = w1.shape[0], w2.shape[0], w3.shape[0]
    w1t = w1.T
    z = _pre_conv_pallas(table, w1t, b1)
    gathered = z[gidx.reshape(-1)]
    sk = s * k
    rows_target = 2048 if sk >= 2048 else sk
    sb = max(1, rows_target // k)
    while s % sb:
        sb //= 2
    body = functools.partial(_sa_conv_body, k=k)
    out = pl.pallas_call(
        body,
        grid=(s // sb,),
        in_specs=[
            pl.BlockSpec((sb * k, o1), lambda i: (i, 0)),
            pl.BlockSpec((sb, 3), lambda i: (i, 0)),
            pl.BlockSpec((3, o1), lambda i: (0, 0)),
            pl.BlockSpec((o1, o2), lambda i: (0, 0)),
            pl.BlockSpec((1, o2), lambda i: (0, 0)),
            pl.BlockSpec((o2, o3), lambda i: (0, 0)),
            pl.BlockSpec((1, o3), lambda i: (0, 0)),
        ],
        out_specs=pl.BlockSpec((sb, o3), lambda i: (i, 0)),
        out_shape=jax.ShapeDtypeStruct((s, o3), jnp.float32),
    )(gathered, new_xyz_t, w1t[cp:cp + 3], w2.T, b2[None], w3.T, b3[None])
    return out


def _conv_bn_relu(x, W, bias, gamma, beta):
    shp = (1, -1) + (1,) * (x.ndim - 2)
    y = jnp.einsum('oc,bc...->bo...', W, x) + bias.reshape(shp)
    y = y / jnp.sqrt(1.0 + BN_EPS)
    y = y * gamma.reshape(shp) + beta.reshape(shp)
    return jax.nn.relu(y)


def _sa_msg(xyz, points, new_xyz, radii, nsamples, branch_params):
    xyz_t = jnp.transpose(xyz, (0, 2, 1))
    points_t = None if points is None else jnp.transpose(points, (0, 2, 1))
    outs = []
    gidx_both = _ball_query2(radii, nsamples, xyz_t[0], new_xyz[0])
    table = jnp.concatenate([points_t[0], xyz_t[0]], axis=1)
    for k, convs, gidx in zip(nsamples, branch_params, gidx_both):
        out = _sa_branch_pallas(table, gidx, new_xyz[0], convs, k)
        outs.append(out.T[None])
    return jnp.transpose(new_xyz, (0, 2, 1)), jnp.concatenate(outs, axis=1)


def _fp_body(*refs, n_convs, has_p1):
    if has_p1:
        (x1_ref, x2t_ref, p2_ref, p1_ref), rest = refs[:4], refs[4:]
    else:
        (x1_ref, x2t_ref, p2_ref), rest = refs[:3], refs[3:]
        p1_ref = None
    wb_refs, o_ref = rest[:-1], rest[-1]
    x1 = x1_ref[...]
    x2t = x2t_ref[...]
    qb = x1.shape[0]
    s2 = x2t.shape[1]
    qn = jnp.sum(x1 * x1, axis=1, keepdims=True)
    xn = jnp.sum(x2t * x2t, axis=0, keepdims=True)
    d = qn + xn - 2.0 * jax.lax.dot(x1, x2t, preferred_element_type=jnp.float32)
    iota = jax.lax.broadcasted_iota(jnp.int32, (qb, s2), 1)
    acc = jnp.zeros((qb, s2), jnp.float32)
    recips = []
    onehots = []
    for _ in range(3):
        m = jnp.min(d, axis=1, keepdims=True)
        jmin = jnp.min(jnp.where(d == m, iota, s2), axis=1, keepdims=True)
        oh = (iota == jmin)
        onehots.append(oh)
        recips.append(1.0 / (m + 1e-8))
        d = jnp.where(oh, 1e30, d)
    norm = recips[0] + recips[1] + recips[2]
    for oh, rc in zip(onehots, recips):
        acc = acc + jnp.where(oh, rc / norm, 0.0)
    h = jax.lax.dot(acc, p2_ref[...], preferred_element_type=jnp.float32)
    if p1_ref is not None:
        h = jnp.concatenate([p1_ref[...], h], axis=1)
    for i in range(n_convs):
        h = jax.lax.dot(h, wb_refs[2 * i][...], preferred_element_type=jnp.float32)
        h = jnp.maximum(h + wb_refs[2 * i + 1][...], 0.0)
    o_ref[...] = h


def _fp_pallas(x1t, x2t, p1, p2, convs):
    s1 = x1t.shape[0]
    s2 = x2t.shape[0]
    c2 = p2.shape[1]
    qb = min(256, s1)
    wbs = [_fold_bn(p) for p in convs]
    n_convs = len(wbs)
    cout = wbs[-1][0].shape[0]
    in_specs = [
        pl.BlockSpec((qb, 3), lambda i: (i, 0)),
        pl.BlockSpec((3, s2), lambda i: (0, 0)),
        pl.BlockSpec((s2, c2), lambda i: (0, 0)),
    ]
    args = [x1t, x2t.T, p2]
    if p1 is not None:
        in_specs.append(pl.BlockSpec((qb, p1.shape[1]), lambda i: (i, 0)))
        args.append(p1)
    for (w, b) in wbs:
        in_specs.append(pl.BlockSpec(w.T.shape, lambda i: (0, 0)))
        in_specs.append(pl.BlockSpec((1, w.shape[0]), lambda i: (0, 0)))
        args.append(w.T)
        args.append(b[None])
    body = functools.partial(_fp_body, n_convs=n_convs, has_p1=p1 is not None)
    return pl.pallas_call(
        body,
        grid=(s1 // qb,),
        in_specs=in_specs,
        out_specs=pl.BlockSpec((qb, cout), lambda i: (i, 0)),
        out_shape=jax.ShapeDtypeStruct((s1, cout), jnp.float32),
    )(*args)


def _fp_module(xyz1, xyz2, points1, points2, convs):
    x1 = jnp.transpose(xyz1, (0, 2, 1))
    x2 = jnp.transpose(xyz2, (0, 2, 1))
    p2 = jnp.transpose(points2, (0, 2, 1))
    dists = _sqdist(x1, x2)
    idx = jnp.argsort(dists, axis=-1)[:, :, :3]
    d3 = jnp.take_along_axis(dists, idx, axis=-1)
    dist_recip = 1.0 / (d3 + 1e-8)
    weight = dist_recip / jnp.sum(dist_recip, axis=2, keepdims=True)
    interp = jnp.sum(_index_points(p2, idx) * weight[..., None], axis=2)
    if points1 is not None:
        newp = jnp.concatenate([jnp.transpose(points1, (0, 2, 1)), interp], axis=-1)
    else:
        newp = interp
    g = jnp.transpose(newp, (0, 2, 1))
    for p in convs:
        g = _conv_bn_relu(g, *p)
    return g


def _head_pallas(h, W2, b2):
    b, c, n = h.shape

    def kern(h_ref, w_ref, b_ref, o_ref):
        hh = h_ref[0]
        logits = jnp.dot(w_ref[...], hh, preferred_element_type=jnp.float32)
        logits = logits + b_ref[...][:, :1]
        m = jnp.max(logits, axis=0, keepdims=True)
        z = logits - m
        lse = jnp.log(jnp.sum(jnp.exp(z), axis=0, keepdims=True))
        o_ref[0] = z - lse

    blk = 2048
    out = pl.pallas_call(
        kern,
        grid=(n // blk,),
        in_specs=[
            pl.BlockSpec((1, c, blk), lambda i: (0, 0, i)),
            pl.BlockSpec((NUM_CLASSES, c), lambda i: (0, 0)),
            pl.BlockSpec((NUM_CLASSES, 1), lambda i: (0, 0)),
        ],
        out_specs=pl.BlockSpec((1, NUM_CLASSES, blk), lambda i: (0, 0, i)),
        out_shape=jax.ShapeDtypeStruct((b, NUM_CLASSES, n), jnp.float32),
    )(h, W2, b2.reshape(NUM_CLASSES, 1))
    return out


@jax.jit
def kernel(xyz, colors, sa_params, fp_params, head1, head2):
    del colors
    mean = jnp.mean(xyz, axis=2, keepdims=True)
    std = jnp.std(xyz, axis=2, keepdims=True, ddof=1)
    std = jnp.where(std == 0, 1e-8, std)
    x = (xyz - mean) / std
    l0_xyz, l0_points = x, x
    x_norm_t = x[0].T
    fps_idx = _fps_chain(x_norm_t)
    layers = []
    cur_xyz, cur_pts = l0_xyz, l0_points
    for cfg, prm, fidx in zip(SA_CFG, sa_params, fps_idx):
        new_xyz = x_norm_t[fidx][None]
        cur_xyz, cur_pts = _sa_msg(cur_xyz, cur_pts, new_xyz, cfg[1], cfg[2], prm)
        layers.append((cur_xyz, cur_pts))
    (l1_xyz, l1_points), (l2_xyz, l2_points), (l3_xyz, l3_points), (l4_xyz, l4_points) = layers

    def fp(xyz1, xyz2, points1, points2, convs):
        p1 = None if points1 is None else points1[0].T
        out = _fp_pallas(xyz1[0].T, xyz2[0].T, p1, points2[0].T, convs)
        return out.T[None]

    l3_points = fp(l3_xyz, l4_xyz, l3_points, l4_points, fp_params[0])
    l2_points = fp(l2_xyz, l3_xyz, l2_points, l3_points, fp_params[1])
    l1_points = fp(l1_xyz, l2_xyz, l1_points, l2_points, fp_params[2])
    l0_points = fp(l0_xyz, l1_xyz, None, l1_points, fp_params[3])
    h = _conv_bn_relu(l0_points, *head1)
    W2, b2 = head2
    logits = _head_pallas(h, W2, b2)
    return jnp.transpose(logits, (0, 2, 1))


# speedup vs baseline: 1.0264x; 1.0010x over previous
import functools

import jax
import jax.numpy as jnp
from jax.experimental import pallas as pl
from jax.experimental.pallas import tpu as pltpu

BN_EPS = 1e-5
NUM_CLASSES = 13
SA_CFG = [
    (4096, (0.1, 0.2), (32, 64)),
    (2048, (0.2, 0.4), (32, 64)),
    (512, (0.4, 0.8), (32, 64)),
    (128, (0.8, 1.6), (32, 64)),
]


def _sqdist(src, dst):
    return (jnp.sum(src ** 2, -1)[:, :, None] + jnp.sum(dst ** 2, -1)[:, None, :]
            - 2.0 * jnp.einsum('bnc,bmc->bnm', src, dst))


def _index_points(points, idx):
    b = points.shape[0]
    bidx = jnp.arange(b).reshape((b,) + (1,) * (idx.ndim - 1))
    return points[bidx, idx]


_FPS_STAGES = (4096, 2048, 512, 128)


def _fps_stage_body(xyz_ref, o_ref, *, S):
    R = xyz_ref.shape[1]
    n = R * 128
    xr = xyz_ref[0]
    yr = xyz_ref[1]
    zr = xyz_ref[2]
    gi = (jax.lax.broadcasted_iota(jnp.int32, (R, 128), 0) * 128
          + jax.lax.broadcasted_iota(jnp.int32, (R, 128), 1))
    RS = S // 128
    gs = (jax.lax.broadcasted_iota(jnp.int32, (RS, 128), 0) * 128
          + jax.lax.broadcasted_iota(jnp.int32, (RS, 128), 1))

    def body(i, state):
        dist, far, idx_out = state
        sel = gi == far
        cx = jnp.sum(jnp.where(sel, xr, 0.0))
        cy = jnp.sum(jnp.where(sel, yr, 0.0))
        cz = jnp.sum(jnp.where(sel, zr, 0.0))
        dx = xr - cx
        dy = yr - cy
        dz = zr - cz
        d = dx * dx + dy * dy + dz * dz
        dist = jnp.minimum(dist, d)
        m = jnp.max(dist)
        far_new = jnp.min(jnp.where(dist == m, gi, n))
        idx_out = jnp.where(gs == i, far, idx_out)
        return dist, far_new, idx_out

    init = (jnp.full((R, 128), 1e10, jnp.float32), jnp.int32(0),
            jnp.zeros((RS, 128), jnp.int32))
    _, _, idx_out = jax.lax.fori_loop(0, S, body, init)
    o_ref[...] = idx_out


def _fps_chain(xyz_norm_t):
    cur = xyz_norm_t
    idx = None
    out = []
    for S in _FPS_STAGES:
        n = cur.shape[0]
        x3 = cur.T.reshape(3, n // 128, 128)
        loc = pl.pallas_call(
            functools.partial(_fps_stage_body, S=S),
            out_shape=jax.ShapeDtypeStruct((S // 128, 128), jnp.int32),
        )(x3).reshape(-1)
        gidx = loc if idx is None else idx[loc]
        out.append(gidx)
        cur = cur[loc]
        idx = gidx
    return tuple(out)


def _ball_query_body(q_ref, x_ref, o1_ref, o2_ref, c_scr, rks):
    x3 = x_ref[...]
    q = q_ref[...]
    n = x3.shape[1]
    qb = q.shape[0]
    qn = jnp.sum(q * q, axis=1, keepdims=True)
    xn = jnp.sum(x3 * x3, axis=0, keepdims=True)
    dist = qn + xn - 2.0 * jax.lax.dot(q, x3, preferred_element_type=jnp.float32)
    tri = (jax.lax.broadcasted_iota(jnp.int32, (128, 128), 0)
           <= jax.lax.broadcasted_iota(jnp.int32, (128, 128), 1)).astype(jnp.float32)
    for (radius, k, oref) in ((rks[0][0], rks[0][1], o1_ref), (rks[1][0], rks[1][1], o2_ref)):
        maskf = jnp.where(dist <= radius * radius, 1.0, 0.0)
        off = jnp.zeros((qb, 1), jnp.float32)
        for m in range(n // 128):
            cm = jax.lax.dot(maskf[:, m * 128:(m + 1) * 128], tri,
                             preferred_element_type=jnp.float32)
            c_scr[:, m * 128:(m + 1) * 128] = cm + off
            off = off + cm[:, 127:128]
        c = c_scr[...]
        cols = []
        for r in range(k):
            cols.append(jnp.sum(jnp.where(c < float(r + 1), 1.0, 0.0), axis=1,
                                keepdims=True))
        cnt = jnp.concatenate(cols, axis=1)
        first = cols[0]
        oref[...] = jnp.where(cnt >= float(n), first, cnt).astype(jnp.int32)


def _ball_query2(radii, nsamples, xyz_t, new_xyz_t):
    n = xyz_t.shape[0]
    s = new_xyz_t.shape[0]
    qb = min(256, s)
    k1, k2 = nsamples
    body = functools.partial(_ball_query_body, rks=((radii[0], k1), (radii[1], k2)))
    o1, o2 = pl.pallas_call(
        body,
        grid=(s // qb,),
        in_specs=[
            pl.BlockSpec((qb, 3), lambda i: (i, 0)),
            pl.BlockSpec((3, n), lambda i: (0, 0)),
        ],
        out_specs=[
            pl.BlockSpec((qb, k1), lambda i: (i, 0)),
            pl.BlockSpec((qb, k2), lambda i: (i, 0)),
        ],
        out_shape=[
            jax.ShapeDtypeStruct((s, k1), jnp.int32),
            jax.ShapeDtypeStruct((s, k2), jnp.int32),
        ],
        scratch_shapes=[pltpu.VMEM((qb, n), jnp.float32)],
    )(new_xyz_t, xyz_t.T)
    return o1, o2


def _fold_bn(p):
    W, b, gamma, beta = p
    s = gamma / jnp.sqrt(1.0 + BN_EPS)
    return W * s[:, None], b * s + beta


def _pre_conv_body(t_ref, w_ref, b_ref, o_ref):
    o_ref[...] = (jax.lax.dot(t_ref[...], w_ref[...],
                              preferred_element_type=jnp.float32) + b_ref[...])


def _pre_conv_pallas(table, w1t, b1):
    n, c = table.shape
    o1 = w1t.shape[1]
    rb = min(1024, n)
    return pl.pallas_call(
        _pre_conv_body,
        grid=(n // rb,),
        in_specs=[
            pl.BlockSpec((rb, c), lambda i: (i, 0)),
            pl.BlockSpec((c, o1), lambda i: (0, 0)),
            pl.BlockSpec((1, o1), lambda i: (0, 0)),
        ],
        out_specs=pl.BlockSpec((rb, o1), lambda i: (i, 0)),
        out_shape=jax.ShapeDtypeStruct((n, o1), jnp.float32),
    )(table, w1t, b1[None])


def _sa_conv_body(g_ref, q_ref, wx_ref, w2_ref, b2_ref, w3_ref, b3_ref, o_ref,
                  *, k):
    sb = q_ref.shape[0]
    off = jax.lax.dot(q_ref[...], wx_ref[...],
                      preferred_element_type=jnp.float32)
    h = g_ref[...].reshape(sb, k, -1) - off[:, None, :]
    h = jnp.maximum(h, 0.0).reshape(sb * k, -1)
    h = jax.lax.dot(h, w2_ref[...], preferred_element_type=jnp.float32) + b2_ref[...]
    h = jnp.maximum(h, 0.0)
    h = jax.lax.dot(h, w3_ref[...], preferred_element_type=jnp.float32) + b3_ref[...]
    h = jnp.maximum(h, 0.0)
    o_ref[...] = jnp.max(h.reshape(sb, k, -1), axis=1)


def _sa_branch_pallas(table, gidx, new_xyz_t, convs, k):
    s = gidx.shape[0]
    cp = table.shape[1] - 3
    (w1, b1), (w2, b2), (w3, b3) = (_fold_bn(p) for p in convs)
    o1, o2, o3 = w1.shape[0], w2.shape[0], w3.shape[0]
    w1t = w1.T
    z = _pre_conv_pallas(table, w1t, b1)
    gathered = z[gidx.reshape(-1)]
    sk = s * k
    rows_target = 2048 if sk >= 2048 else sk
    sb = max(1, rows_target // k)
    while s % sb:
        sb //= 2
    body = functools.partial(_sa_conv_body, k=k)
    out = pl.pallas_call(
        body,
        grid=(s // sb,),
        in_specs=[
            pl.BlockSpec((sb * k, o1), lambda i: (i, 0)),
            pl.BlockSpec((sb, 3), lambda i: (i, 0)),
            pl.BlockSpec((3, o1), lambda i: (0, 0)),
            pl.BlockSpec((o1, o2), lambda i: (0, 0)),
            pl.BlockSpec((1, o2), lambda i: (0, 0)),
            pl.BlockSpec((o2, o3), lambda i: (0, 0)),
            pl.BlockSpec((1, o3), lambda i: (0, 0)),
        ],
        out_specs=pl.BlockSpec((sb, o3), lambda i: (i, 0)),
        out_shape=jax.ShapeDtypeStruct((s, o3), jnp.float32),
    )(gathered, new_xyz_t, w1t[cp:cp + 3], w2.T, b2[None], w3.T, b3[None])
    return out


def _conv_bn_relu(x, W, bias, gamma, beta):
    shp = (1, -1) + (1,) * (x.ndim - 2)
    y = jnp.einsum('oc,bc...->bo...', W, x) + bias.reshape(shp)
    y = y / jnp.sqrt(1.0 + BN_EPS)
    y = y * gamma.reshape(shp) + beta.reshape(shp)
    return jax.nn.relu(y)


def _sa_msg(xyz, points, new_xyz, radii, nsamples, branch_params):
    xyz_t = jnp.transpose(xyz, (0, 2, 1))
    points_t = None if points is None else jnp.transpose(points, (0, 2, 1))
    outs = []
    gidx_both = _ball_query2(radii, nsamples, xyz_t[0], new_xyz[0])
    table = jnp.concatenate([points_t[0], xyz_t[0]], axis=1)
    for k, convs, gidx in zip(nsamples, branch_params, gidx_both):
        out = _sa_branch_pallas(table, gidx, new_xyz[0], convs, k)
        outs.append(out.T[None])
    return jnp.transpose(new_xyz, (0, 2, 1)), jnp.concatenate(outs, axis=1)


def _fp_body(*refs, n_convs, has_p1):
    if has_p1:
        (x1_ref, x2t_ref, p2_ref, p1_ref), rest = refs[:4], refs[4:]
    else:
        (x1_ref, x2t_ref, p2_ref), rest = refs[:3], refs[3:]
        p1_ref = None
    wb_refs, o_ref = rest[:-1], rest[-1]
    x1 = x1_ref[...]
    x2t = x2t_ref[...]
    qb = x1.shape[0]
    s2 = x2t.shape[1]
    qn = jnp.sum(x1 * x1, axis=1, keepdims=True)
    xn = jnp.sum(x2t * x2t, axis=0, keepdims=True)
    d = qn + xn - 2.0 * jax.lax.dot(x1, x2t, preferred_element_type=jnp.float32)
    iota = jax.lax.broadcasted_iota(jnp.int32, (qb, s2), 1)
    acc = jnp.zeros((qb, s2), jnp.float32)
    recips = []
    onehots = []
    for _ in range(3):
        m = jnp.min(d, axis=1, keepdims=True)
        jmin = jnp.min(jnp.where(d == m, iota, s2), axis=1, keepdims=True)
        oh = (iota == jmin)
        onehots.append(oh)
        recips.append(1.0 / (m + 1e-8))
        d = jnp.where(oh, 1e30, d)
    norm = recips[0] + recips[1] + recips[2]
    for oh, rc in zip(onehots, recips):
        acc = acc + jnp.where(oh, rc / norm, 0.0)
    h = jax.lax.dot(acc, p2_ref[...], preferred_element_type=jnp.float32)
    if p1_ref is not None:
        h = jnp.concatenate([p1_ref[...], h], axis=1)
    for i in range(n_convs):
        h = jax.lax.dot(h, wb_refs[2 * i][...], preferred_element_type=jnp.float32)
        h = jnp.maximum(h + wb_refs[2 * i + 1][...], 0.0)
    o_ref[...] = h


def _fp_pallas(x1t, x2t, p1, p2, convs):
    s1 = x1t.shape[0]
    s2 = x2t.shape[0]
    c2 = p2.shape[1]
    qb = min(256, s1)
    wbs = [_fold_bn(p) for p in convs]
    n_convs = len(wbs)
    cout = wbs[-1][0].shape[0]
    in_specs = [
        pl.BlockSpec((qb, 3), lambda i: (i, 0)),
        pl.BlockSpec((3, s2), lambda i: (0, 0)),
        pl.BlockSpec((s2, c2), lambda i: (0, 0)),
    ]
    args = [x1t, x2t.T, p2]
    if p1 is not None:
        in_specs.append(pl.BlockSpec((qb, p1.shape[1]), lambda i: (i, 0)))
        args.append(p1)
    for (w, b) in wbs:
        in_specs.append(pl.BlockSpec(w.T.shape, lambda i: (0, 0)))
        in_specs.append(pl.BlockSpec((1, w.shape[0]), lambda i: (0, 0)))
        args.append(w.T)
        args.append(b[None])
    body = functools.partial(_fp_body, n_convs=n_convs, has_p1=p1 is not None)
    return pl.pallas_call(
        body,
        grid=(s1 // qb,),
        in_specs=in_specs,
        out_specs=pl.BlockSpec((qb, cout), lambda i: (i, 0)),
        out_shape=jax.ShapeDtypeStruct((s1, cout), jnp.float32),
    )(*args)


def _fp_module(xyz1, xyz2, points1, points2, convs):
    x1 = jnp.transpose(xyz1, (0, 2, 1))
    x2 = jnp.transpose(xyz2, (0, 2, 1))
    p2 = jnp.transpose(points2, (0, 2, 1))
    dists = _sqdist(x1, x2)
    idx = jnp.argsort(dists, axis=-1)[:, :, :3]
    d3 = jnp.take_along_axis(dists, idx, axis=-1)
    dist_recip = 1.0 / (d3 + 1e-8)
    weight = dist_recip / jnp.sum(dist_recip, axis=2, keepdims=True)
    interp = jnp.sum(_index_points(p2, idx) * weight[..., None], axis=2)
    if points1 is not None:
        newp = jnp.concatenate([jnp.transpose(points1, (0, 2, 1)), interp], axis=-1)
    else:
        newp = interp
    g = jnp.transpose(newp, (0, 2, 1))
    for p in convs:
        g = _conv_bn_relu(g, *p)
    return g


def _head_pallas(h, head1, W2, b2):
    b, c, n = h.shape
    w1, b1 = _fold_bn(head1)
    c1 = w1.shape[0]

    def kern(h_ref, w1_ref, b1_ref, w_ref, b_ref, o_ref):
        hh = jnp.dot(w1_ref[...], h_ref[0], preferred_element_type=jnp.float32)
        hh = jnp.maximum(hh + b1_ref[...][:, :1], 0.0)
        logits = jnp.dot(w_ref[...], hh, preferred_element_type=jnp.float32)
        logits = logits + b_ref[...][:, :1]
        m = jnp.max(logits, axis=0, keepdims=True)
        z = logits - m
        lse = jnp.log(jnp.sum(jnp.exp(z), axis=0, keepdims=True))
        o_ref[0] = z - lse

    blk = 2048
    out = pl.pallas_call(
        kern,
        grid=(n // blk,),
        in_specs=[
            pl.BlockSpec((1, c, blk), lambda i: (0, 0, i)),
            pl.BlockSpec((c1, c), lambda i: (0, 0)),
            pl.BlockSpec((c1, 1), lambda i: (0, 0)),
            pl.BlockSpec((NUM_CLASSES, c1), lambda i: (0, 0)),
            pl.BlockSpec((NUM_CLASSES, 1), lambda i: (0, 0)),
        ],
        out_specs=pl.BlockSpec((1, NUM_CLASSES, blk), lambda i: (0, 0, i)),
        out_shape=jax.ShapeDtypeStruct((b, NUM_CLASSES, n), jnp.float32),
    )(h, w1, b1.reshape(c1, 1), W2, b2.reshape(NUM_CLASSES, 1))
    return out


@jax.jit
def kernel(xyz, colors, sa_params, fp_params, head1, head2):
    del colors
    mean = jnp.mean(xyz, axis=2, keepdims=True)
    std = jnp.std(xyz, axis=2, keepdims=True, ddof=1)
    std = jnp.where(std == 0, 1e-8, std)
    x = (xyz - mean) / std
    l0_xyz, l0_points = x, x
    x_norm_t = x[0].T
    fps_idx = _fps_chain(x_norm_t)
    layers = []
    cur_xyz, cur_pts = l0_xyz, l0_points
    for cfg, prm, fidx in zip(SA_CFG, sa_params, fps_idx):
        new_xyz = x_norm_t[fidx][None]
        cur_xyz, cur_pts = _sa_msg(cur_xyz, cur_pts, new_xyz, cfg[1], cfg[2], prm)
        layers.append((cur_xyz, cur_pts))
    (l1_xyz, l1_points), (l2_xyz, l2_points), (l3_xyz, l3_points), (l4_xyz, l4_points) = layers

    def fp(xyz1, xyz2, points1, points2, convs):
        p1 = None if points1 is None else points1[0].T
        out = _fp_pallas(xyz1[0].T, xyz2[0].T, p1, points2[0].T, convs)
        return out.T[None]

    l3_points = fp(l3_xyz, l4_xyz, l3_points, l4_points, fp_params[0])
    l2_points = fp(l2_xyz, l3_xyz, l2_points, l3_points, fp_params[1])
    l1_points = fp(l1_xyz, l2_xyz, l1_points, l2_points, fp_params[2])
    l0_points = fp(l0_xyz, l1_xyz, None, l1_points, fp_params[3])
    W2, b2 = head2
    logits = _head_pallas(l0_points, head1, W2, b2)
    return jnp.transpose(logits, (0, 2, 1))


# speedup vs baseline: 1.0267x; 1.0002x over previous
import functools

import jax
import jax.numpy as jnp
from jax.experimental import pallas as pl
from jax.experimental.pallas import tpu as pltpu

BN_EPS = 1e-5
NUM_CLASSES = 13
SA_CFG = [
    (4096, (0.1, 0.2), (32, 64)),
    (2048, (0.2, 0.4), (32, 64)),
    (512, (0.4, 0.8), (32, 64)),
    (128, (0.8, 1.6), (32, 64)),
]


_FPS_STAGES = (4096, 2048, 512, 128)


def _fps_stage_body(xyz_ref, o_ref, *, S):
    R = xyz_ref.shape[1]
    n = R * 128
    xr = xyz_ref[0]
    yr = xyz_ref[1]
    zr = xyz_ref[2]
    gi = (jax.lax.broadcasted_iota(jnp.int32, (R, 128), 0) * 128
          + jax.lax.broadcasted_iota(jnp.int32, (R, 128), 1))
    RS = S // 128
    gs = (jax.lax.broadcasted_iota(jnp.int32, (RS, 128), 0) * 128
          + jax.lax.broadcasted_iota(jnp.int32, (RS, 128), 1))

    def body(i, state):
        dist, far, idx_out = state
        sel = gi == far
        cx = jnp.sum(jnp.where(sel, xr, 0.0))
        cy = jnp.sum(jnp.where(sel, yr, 0.0))
        cz = jnp.sum(jnp.where(sel, zr, 0.0))
        dx = xr - cx
        dy = yr - cy
        dz = zr - cz
        d = dx * dx + dy * dy + dz * dz
        dist = jnp.minimum(dist, d)
        m = jnp.max(dist)
        far_new = jnp.min(jnp.where(dist == m, gi, n))
        idx_out = jnp.where(gs == i, far, idx_out)
        return dist, far_new, idx_out

    init = (jnp.full((R, 128), 1e10, jnp.float32), jnp.int32(0),
            jnp.zeros((RS, 128), jnp.int32))
    _, _, idx_out = jax.lax.fori_loop(0, S, body, init)
    o_ref[...] = idx_out


def _fps_chain(xyz_norm_t):
    cur = xyz_norm_t
    idx = None
    out = []
    for S in _FPS_STAGES:
        n = cur.shape[0]
        x3 = cur.T.reshape(3, n // 128, 128)
        loc = pl.pallas_call(
            functools.partial(_fps_stage_body, S=S),
            out_shape=jax.ShapeDtypeStruct((S // 128, 128), jnp.int32),
        )(x3).reshape(-1)
        gidx = loc if idx is None else idx[loc]
        out.append(gidx)
        cur = cur[loc]
        idx = gidx
    return tuple(out)


def _ball_query_body(q_ref, x_ref, o1_ref, o2_ref, c_scr, rks):
    x3 = x_ref[...]
    q = q_ref[...]
    n = x3.shape[1]
    qb = q.shape[0]
    qn = jnp.sum(q * q, axis=1, keepdims=True)
    xn = jnp.sum(x3 * x3, axis=0, keepdims=True)
    dist = qn + xn - 2.0 * jax.lax.dot(q, x3, preferred_element_type=jnp.float32)
    tri = (jax.lax.broadcasted_iota(jnp.int32, (128, 128), 0)
           <= jax.lax.broadcasted_iota(jnp.int32, (128, 128), 1)).astype(jnp.float32)
    for (radius, k, oref) in ((rks[0][0], rks[0][1], o1_ref), (rks[1][0], rks[1][1], o2_ref)):
        maskf = jnp.where(dist <= radius * radius, 1.0, 0.0)
        off = jnp.zeros((qb, 1), jnp.float32)
        for m in range(n // 128):
            cm = jax.lax.dot(maskf[:, m * 128:(m + 1) * 128], tri,
                             preferred_element_type=jnp.float32)
            c_scr[:, m * 128:(m + 1) * 128] = cm + off
            off = off + cm[:, 127:128]
        c = c_scr[...]
        cols = []
        for r in range(k):
            cols.append(jnp.sum(jnp.where(c < float(r + 1), 1.0, 0.0), axis=1,
                                keepdims=True))
        cnt = jnp.concatenate(cols, axis=1)
        first = cols[0]
        oref[...] = jnp.where(cnt >= float(n), first, cnt).astype(jnp.int32)


def _ball_query2(radii, nsamples, xyz_t, new_xyz_t):
    n = xyz_t.shape[0]
    s = new_xyz_t.shape[0]
    qb = min(256, s)
    k1, k2 = nsamples
    body = functools.partial(_ball_query_body, rks=((radii[0], k1), (radii[1], k2)))
    o1, o2 = pl.pallas_call(
        body,
        grid=(s // qb,),
        in_specs=[
            pl.BlockSpec((qb, 3), lambda i: (i, 0)),
            pl.BlockSpec((3, n), lambda i: (0, 0)),
        ],
        out_specs=[
            pl.BlockSpec((qb, k1), lambda i: (i, 0)),
            pl.BlockSpec((qb, k2), lambda i: (i, 0)),
        ],
        out_shape=[
            jax.ShapeDtypeStruct((s, k1), jnp.int32),
            jax.ShapeDtypeStruct((s, k2), jnp.int32),
        ],
        scratch_shapes=[pltpu.VMEM((qb, n), jnp.float32)],
    )(new_xyz_t, xyz_t.T)
    return o1, o2


def _fold_bn(p):
    W, b, gamma, beta = p
    s = gamma / jnp.sqrt(1.0 + BN_EPS)
    return W * s[:, None], b * s + beta


def _pre_conv_body(t_ref, w_ref, b_ref, o_ref):
    o_ref[...] = (jax.lax.dot(t_ref[...], w_ref[...],
                              preferred_element_type=jnp.float32) + b_ref[...])


def _pre_conv_pallas(table, w1t, b1):
    n, c = table.shape
    o1 = w1t.shape[1]
    rb = min(1024, n)
    return pl.pallas_call(
        _pre_conv_body,
        grid=(n // rb,),
        in_specs=[
            pl.BlockSpec((rb, c), lambda i: (i, 0)),
            pl.BlockSpec((c, o1), lambda i: (0, 0)),
            pl.BlockSpec((1, o1), lambda i: (0, 0)),
        ],
        out_specs=pl.BlockSpec((rb, o1), lambda i: (i, 0)),
        out_shape=jax.ShapeDtypeStruct((n, o1), jnp.float32),
    )(table, w1t, b1[None])


def _sa_conv_body(g_ref, q_ref, wx_ref, w2_ref, b2_ref, w3_ref, b3_ref, o_ref,
                  *, k):
    sb = q_ref.shape[0]
    off = jax.lax.dot(q_ref[...], wx_ref[...],
                      preferred_element_type=jnp.float32)
    h = g_ref[...].reshape(sb, k, -1) - off[:, None, :]
    h = jnp.maximum(h, 0.0).reshape(sb * k, -1)
    h = jax.lax.dot(h, w2_ref[...], preferred_element_type=jnp.float32) + b2_ref[...]
    h = jnp.maximum(h, 0.0)
    h = jax.lax.dot(h, w3_ref[...], preferred_element_type=jnp.float32) + b3_ref[...]
    h = jnp.maximum(h, 0.0)
    o_ref[...] = jnp.max(h.reshape(sb, k, -1), axis=1)


def _sa_branch_pallas(table, gidx, new_xyz_t, convs, k):
    s = gidx.shape[0]
    cp = table.shape[1] - 3
    (w1, b1), (w2, b2), (w3, b3) = (_fold_bn(p) for p in convs)
    o1, o2, o3 = w1.shape[0], w2.shape[0], w3.shape[0]
    w1t = w1.T
    z = _pre_conv_pallas(table, w1t, b1)
    gathered = z[gidx.reshape(-1)]
    sk = s * k
    rows_target = 2048 if sk >= 2048 else sk
    sb = max(1, rows_target // k)
    while s % sb:
        sb //= 2
    body = functools.partial(_sa_conv_body, k=k)
    out = pl.pallas_call(
        body,
        grid=(s // sb,),
        in_specs=[
            pl.BlockSpec((sb * k, o1), lambda i: (i, 0)),
            pl.BlockSpec((sb, 3), lambda i: (i, 0)),
            pl.BlockSpec((3, o1), lambda i: (0, 0)),
            pl.BlockSpec((o1, o2), lambda i: (0, 0)),
            pl.BlockSpec((1, o2), lambda i: (0, 0)),
            pl.BlockSpec((o2, o3), lambda i: (0, 0)),
            pl.BlockSpec((1, o3), lambda i: (0, 0)),
        ],
        out_specs=pl.BlockSpec((sb, o3), lambda i: (i, 0)),
        out_shape=jax.ShapeDtypeStruct((s, o3), jnp.float32),
    )(gathered, new_xyz_t, w1t[cp:cp + 3], w2.T, b2[None], w3.T, b3[None])
    return out


def _sa_msg(xyz, points, new_xyz, radii, nsamples, branch_params):
    xyz_t = jnp.transpose(xyz, (0, 2, 1))
    points_t = None if points is None else jnp.transpose(points, (0, 2, 1))
    outs = []
    gidx_both = _ball_query2(radii, nsamples, xyz_t[0], new_xyz[0])
    table = jnp.concatenate([points_t[0], xyz_t[0]], axis=1)
    for k, convs, gidx in zip(nsamples, branch_params, gidx_both):
        out = _sa_branch_pallas(table, gidx, new_xyz[0], convs, k)
        outs.append(out.T[None])
    return jnp.transpose(new_xyz, (0, 2, 1)), jnp.concatenate(outs, axis=1)


def _fp_body(*refs, n_convs, has_p1):
    if has_p1:
        (x1_ref, x2t_ref, p2_ref, p1_ref), rest = refs[:4], refs[4:]
    else:
        (x1_ref, x2t_ref, p2_ref), rest = refs[:3], refs[3:]
        p1_ref = None
    wb_refs, o_ref = rest[:-1], rest[-1]
    x1 = x1_ref[...]
    x2t = x2t_ref[...]
    qb = x1.shape[0]
    s2 = x2t.shape[1]
    qn = jnp.sum(x1 * x1, axis=1, keepdims=True)
    xn = jnp.sum(x2t * x2t, axis=0, keepdims=True)
    d = qn + xn - 2.0 * jax.lax.dot(x1, x2t, preferred_element_type=jnp.float32)
    iota = jax.lax.broadcasted_iota(jnp.int32, (qb, s2), 1)
    acc = jnp.zeros((qb, s2), jnp.float32)
    recips = []
    onehots = []
    for _ in range(3):
        m = jnp.min(d, axis=1, keepdims=True)
        jmin = jnp.min(jnp.where(d == m, iota, s2), axis=1, keepdims=True)
        oh = (iota == jmin)
        onehots.append(oh)
        recips.append(1.0 / (m + 1e-8))
        d = jnp.where(oh, 1e30, d)
    norm = recips[0] + recips[1] + recips[2]
    for oh, rc in zip(onehots, recips):
        acc = acc + jnp.where(oh, rc / norm, 0.0)
    h = jax.lax.dot(acc, p2_ref[...], preferred_element_type=jnp.float32)
    if p1_ref is not None:
        h = jnp.concatenate([p1_ref[...], h], axis=1)
    for i in range(n_convs):
        h = jax.lax.dot(h, wb_refs[2 * i][...], preferred_element_type=jnp.float32)
        h = jnp.maximum(h + wb_refs[2 * i + 1][...], 0.0)
    o_ref[...] = h


def _fp_pallas(x1t, x2t, p1, p2, convs):
    s1 = x1t.shape[0]
    s2 = x2t.shape[0]
    c2 = p2.shape[1]
    qb = min(256, s1)
    wbs = [_fold_bn(p) for p in convs]
    n_convs = len(wbs)
    cout = wbs[-1][0].shape[0]
    in_specs = [
        pl.BlockSpec((qb, 3), lambda i: (i, 0)),
        pl.BlockSpec((3, s2), lambda i: (0, 0)),
        pl.BlockSpec((s2, c2), lambda i: (0, 0)),
    ]
    args = [x1t, x2t.T, p2]
    if p1 is not None:
        in_specs.append(pl.BlockSpec((qb, p1.shape[1]), lambda i: (i, 0)))
        args.append(p1)
    for (w, b) in wbs:
        in_specs.append(pl.BlockSpec(w.T.shape, lambda i: (0, 0)))
        in_specs.append(pl.BlockSpec((1, w.shape[0]), lambda i: (0, 0)))
        args.append(w.T)
        args.append(b[None])
    body = functools.partial(_fp_body, n_convs=n_convs, has_p1=p1 is not None)
    return pl.pallas_call(
        body,
        grid=(s1 // qb,),
        in_specs=in_specs,
        out_specs=pl.BlockSpec((qb, cout), lambda i: (i, 0)),
        out_shape=jax.ShapeDtypeStruct((s1, cout), jnp.float32),
    )(*args)


def _head_pallas(h, head1, W2, b2):
    b, c, n = h.shape
    w1, b1 = _fold_bn(head1)
    c1 = w1.shape[0]

    def kern(h_ref, w1_ref, b1_ref, w_ref, b_ref, o_ref):
        hh = jnp.dot(w1_ref[...], h_ref[0], preferred_element_type=jnp.float32)
        hh = jnp.maximum(hh + b1_ref[...][:, :1], 0.0)
        logits = jnp.dot(w_ref[...], hh, preferred_element_type=jnp.float32)
        logits = logits + b_ref[...][:, :1]
        m = jnp.max(logits, axis=0, keepdims=True)
        z = logits - m
        lse = jnp.log(jnp.sum(jnp.exp(z), axis=0, keepdims=True))
        o_ref[0] = z - lse

    blk = 2048
    out = pl.pallas_call(
        kern,
        grid=(n // blk,),
        in_specs=[
            pl.BlockSpec((1, c, blk), lambda i: (0, 0, i)),
            pl.BlockSpec((c1, c), lambda i: (0, 0)),
            pl.BlockSpec((c1, 1), lambda i: (0, 0)),
            pl.BlockSpec((NUM_CLASSES, c1), lambda i: (0, 0)),
            pl.BlockSpec((NUM_CLASSES, 1), lambda i: (0, 0)),
        ],
        out_specs=pl.BlockSpec((1, NUM_CLASSES, blk), lambda i: (0, 0, i)),
        out_shape=jax.ShapeDtypeStruct((b, NUM_CLASSES, n), jnp.float32),
    )(h, w1, b1.reshape(c1, 1), W2, b2.reshape(NUM_CLASSES, 1))
    return out


@jax.jit
def kernel(xyz, colors, sa_params, fp_params, head1, head2):
    del colors
    mean = jnp.mean(xyz, axis=2, keepdims=True)
    std = jnp.std(xyz, axis=2, keepdims=True, ddof=1)
    std = jnp.where(std == 0, 1e-8, std)
    x = (xyz - mean) / std
    l0_xyz, l0_points = x, x
    x_norm_t = x[0].T
    fps_idx = _fps_chain(x_norm_t)
    layers = []
    cur_xyz, cur_pts = l0_xyz, l0_points
    for cfg, prm, fidx in zip(SA_CFG, sa_params, fps_idx):
        new_xyz = x_norm_t[fidx][None]
        cur_xyz, cur_pts = _sa_msg(cur_xyz, cur_pts, new_xyz, cfg[1], cfg[2], prm)
        layers.append((cur_xyz, cur_pts))
    (l1_xyz, l1_points), (l2_xyz, l2_points), (l3_xyz, l3_points), (l4_xyz, l4_points) = layers

    def fp(xyz1, xyz2, points1, points2, convs):
        p1 = None if points1 is None else points1[0].T
        out = _fp_pallas(xyz1[0].T, xyz2[0].T, p1, points2[0].T, convs)
        return out.T[None]

    l3_points = fp(l3_xyz, l4_xyz, l3_points, l4_points, fp_params[0])
    l2_points = fp(l2_xyz, l3_xyz, l2_points, l3_points, fp_params[1])
    l1_points = fp(l1_xyz, l2_xyz, l1_points, l2_points, fp_params[2])
    l0_points = fp(l0_xyz, l1_xyz, None, l1_points, fp_params[3])
    W2, b2 = head2
    logits = _head_pallas(l0_points, head1, W2, b2)
    return jnp.transpose(logits, (0, 2, 1))
